# baseline (speedup 1.0000x reference)
import functools

import jax
import jax.numpy as jnp
from jax import lax
from jax.experimental import pallas as pl
from jax.experimental.pallas import tpu as pltpu
from jax.experimental.pallas import tpu_sc as plsc

_NC = 2
_NS = 16
_NW = _NC * _NS
_WIN = 4096
_STRIDE = 5


@functools.lru_cache(maxsize=None)
def _make_sc_agg(n_pad: int, fw: int, e_pad: int):
    assert e_pad % (2 * _WIN) == 0 and _WIN % 16 == 0
    n_win = e_pad // _WIN
    iters = _WIN // 16
    mesh = plsc.VectorSubcoreMesh(core_axis_name="c", subcore_axis_name="s")

    @functools.partial(
        pl.kernel,
        out_type=jax.ShapeDtypeStruct((_NW, n_pad * _STRIDE),
                                      jnp.float32),
        mesh=mesh,
        scratch_types=[
            pltpu.VMEM((n_pad * _STRIDE,), jnp.float32),
            pltpu.VMEM((n_pad * _STRIDE,), jnp.float32),
            pltpu.VMEM((2, _WIN), jnp.int32),
            pltpu.VMEM((2, _WIN), jnp.int32),
        ] + [pltpu.SemaphoreType.DMA] * 4,
        compiler_params=pltpu.CompilerParams(needs_layout_passes=False),
    )
    def agg_kernel(x4_hbm, src_hbm, dst_hbm, out_hbm, table, acc,
                   src_v, dst_v, ss0, ss1, ds0, ds1):
        c = lax.axis_index("c")
        s = lax.axis_index("s")
        wid = s * _NC + c
        ssems = (ss0, ss1)
        dsems = (ds0, ds1)
        pltpu.sync_copy(x4_hbm.at[wid], table)
        pltpu.sync_copy(x4_hbm.at[wid], acc)

        def start(w, b):
            pltpu.async_copy(src_hbm.at[pl.ds(w * _WIN, _WIN)],
                             src_v.at[b], ssems[b])
            pltpu.async_copy(dst_hbm.at[pl.ds(w * _WIN, _WIN)],
                             dst_v.at[b], dsems[b])

        def wait(w, b):
            pltpu.make_async_copy(src_hbm.at[pl.ds(w * _WIN, _WIN)],
                                  src_v.at[b], ssems[b]).wait()
            pltpu.make_async_copy(dst_hbm.at[pl.ds(w * _WIN, _WIN)],
                                  dst_v.at[b], dsems[b]).wait()

        start(0, 0)

        def wpair(p, carry):
            for b in (0, 1):
                w = p * 2 + b
                wait(w, b)
                start(lax.rem(w + 1, n_win), 1 - b)

                @plsc.parallel_loop(0, iters, 1, unroll=8)
                def _(i):
                    s16 = src_v[b, pl.ds(i * 16, 16)]
                    d16 = dst_v[b, pl.ds(i * 16, 16)]
                    gs = [plsc.load_gather(table, (s16 + k,))
                          for k in range(fw)]
                    for k in range(fw):
                        plsc.addupdate_scatter(acc, (d16 + k,), gs[k])
            return carry

        lax.fori_loop(0, n_win // 2, wpair, 0)
        wait(0, 0)
        pltpu.sync_copy(acc, out_hbm.at[wid])

    return agg_kernel


def _dense_layer_body(h_ref, wa_ref, ba_ref, g_ref, be_ref,
                      wb_ref, bb_ref, o_ref):
    t = lax.dot_general(h_ref[...], wa_ref[...], (((1,), (1,)), ((), ())),
                        preferred_element_type=jnp.float32) + ba_ref[...]
    mu = jnp.mean(t, axis=0, keepdims=True)
    d = t - mu
    var = jnp.mean(d * d, axis=0, keepdims=True)
    hn = d * lax.rsqrt(var + 1e-5) * g_ref[...] + be_ref[...]
    hn = jnp.maximum(hn, 0.0)
    h2 = lax.dot_general(hn, wb_ref[...], (((1,), (1,)), ((), ())),
                         preferred_element_type=jnp.float32) + bb_ref[...]
    o_ref[...] = jnp.maximum(h2, 0.0)


def _head_layer_body(h_ref, wa_ref, ba_ref, g_ref, be_ref,
                     wb_ref, bb_ref, wl1_ref, bl1_ref, wl2_ref, bl2_ref,
                     o_ref):
    t = lax.dot_general(h_ref[...], wa_ref[...], (((1,), (1,)), ((), ())),
                        preferred_element_type=jnp.float32) + ba_ref[...]
    mu = jnp.mean(t, axis=0, keepdims=True)
    d = t - mu
    var = jnp.mean(d * d, axis=0, keepdims=True)
    hn = d * lax.rsqrt(var + 1e-5) * g_ref[...] + be_ref[...]
    hn = jnp.maximum(hn, 0.0)
    h2 = lax.dot_general(hn, wb_ref[...], (((1,), (1,)), ((), ())),
                         preferred_element_type=jnp.float32) + bb_ref[...]
    h2 = jnp.maximum(h2, 0.0)
    h3 = lax.dot_general(h2, wl1_ref[...], (((1,), (1,)), ((), ())),
                         preferred_element_type=jnp.float32) + bl1_ref[...]
    h3 = jnp.maximum(h3, 0.0)
    y = jnp.sum(h3 * wl2_ref[...], axis=1, keepdims=True) + bl2_ref[...]
    o_ref[...] = 1.0 / (1.0 + jnp.exp(-y))


def _gin_dense(h, Wa, ba, g, be, Wb, bb, interpret=False):
    n, f = h.shape
    return pl.pallas_call(
        _dense_layer_body,
        out_shape=jax.ShapeDtypeStruct((n, f), jnp.float32),
        interpret=interpret,
    )(h, Wa, ba.reshape(1, -1), g.reshape(1, -1), be.reshape(1, -1),
      Wb, bb.reshape(1, -1))


def _gin_dense_head(h, Wa, ba, g, be, Wb, bb, Wl1, bl1, Wl2, bl2,
                    interpret=False):
    n, f = h.shape
    return pl.pallas_call(
        _head_layer_body,
        out_shape=jax.ShapeDtypeStruct((n, 1), jnp.float32),
        interpret=interpret,
    )(h, Wa, ba.reshape(1, -1), g.reshape(1, -1), be.reshape(1, -1),
      Wb, bb.reshape(1, -1), Wl1, bl1.reshape(1, -1), Wl2,
      bl2.reshape(1, 1))


def kernel(x, edge_index, W1a, b1a, g1, be1, W1b, b1b, W2a, b2a, g2, be2,
           W2b, b2b, Wl1, bl1, Wl2, bl2):
    n, f = x.shape
    fw = f // _NW
    e = edge_index.shape[1]
    e_pad = -(-e // (2 * _WIN)) * (2 * _WIN)
    src = jnp.concatenate([edge_index[0],
                           jnp.zeros((e_pad - e,), jnp.int32)]) * _STRIDE
    dst = jnp.concatenate([edge_index[1],
                           jnp.full((e_pad - e,), n, jnp.int32)]) * _STRIDE

    n_pad = -(-n // 8) * 8
    agg = _make_sc_agg(n_pad, fw, e_pad)

    def run_agg(nodes):
        nodes_p = jnp.pad(nodes, ((0, n_pad - n), (0, 0)))
        x4 = nodes_p.reshape(n_pad, _NW, fw).transpose(1, 0, 2)
        x4 = jnp.pad(x4, ((0, 0), (0, 0), (0, _STRIDE - fw)))
        p = agg(x4.reshape(_NW, n_pad * _STRIDE), src, dst)
        return p.reshape(_NW, n_pad, _STRIDE)[:, :, :fw].transpose(
            1, 0, 2).reshape(n_pad, f)[:n]

    h0 = run_agg(x)
    h1 = _gin_dense(h0, W1a, b1a, g1, be1, W1b, b1b)
    h2 = run_agg(h1)
    return _gin_dense_head(h2, W2a, b2a, g2, be2, W2b, b2b,
                           Wl1, bl1, Wl2, bl2)


# speedup vs baseline: 1.1415x; 1.1415x over previous
import functools

import jax
import jax.numpy as jnp
from jax import lax
from jax.experimental import pallas as pl
from jax.experimental.pallas import tpu as pltpu
from jax.experimental.pallas import tpu_sc as plsc

_NC = 2
_NS = 16
_NW = _NC * _NS
_CH = 128


@functools.lru_cache(maxsize=None)
def _make_sc_agg(n_pad: int, n_feat: int, chunks: int):
    rows_per_tile = n_pad // _NS
    assert rows_per_tile * _NS == n_pad and rows_per_tile % 8 == 0
    mesh = plsc.VectorSubcoreMesh(core_axis_name="c", subcore_axis_name="s")

    @functools.partial(
        pl.kernel,
        out_type=jax.ShapeDtypeStruct((_NC, n_pad, n_feat), jnp.float32),
        mesh=mesh,
        scratch_types=[
            pltpu.VMEM((chunks, _CH), jnp.int32),
            pltpu.VMEM((chunks, _CH), jnp.int32),
            pltpu.VMEM((_CH, n_feat), jnp.float32),
            pltpu.VMEM_SHARED((n_pad, n_feat), jnp.float32),
            pltpu.SemaphoreType.DMA,
        ],
    )
    def agg_kernel(x_hbm, src_hbm, dst_hbm, out_hbm, src_v, dst_v, rows_v,
                   acc, sem):
        c = lax.axis_index("c")
        s = lax.axis_index("s")
        wid = s * _NC + c
        pltpu.sync_copy(src_hbm.at[wid], src_v)
        pltpu.sync_copy(dst_hbm.at[wid], dst_v)
        row0 = s * rows_per_tile
        pltpu.sync_copy(x_hbm.at[pl.ds(row0, rows_per_tile)],
                        acc.at[pl.ds(row0, rows_per_tile)])
        plsc.subcore_barrier()

        def body(j, carry):
            pltpu.async_copy(x_hbm.at[src_v.at[j]], rows_v, sem).wait()
            pltpu.sync_copy(rows_v, acc.at[dst_v.at[j]], add=True)
            return carry

        lax.fori_loop(0, chunks, body, 0)
        plsc.subcore_barrier()
        pltpu.sync_copy(acc.at[pl.ds(row0, rows_per_tile)],
                        out_hbm.at[c, pl.ds(row0, rows_per_tile)])

    return agg_kernel


def _dense_layer_body(x_ref, p0_ref, p1_ref, wa_ref, ba_ref, g_ref, be_ref,
                      wb_ref, bb_ref, o_ref):
    h = p0_ref[...] + p1_ref[...] - x_ref[...]
    t = lax.dot_general(h, wa_ref[...], (((1,), (1,)), ((), ())),
                        preferred_element_type=jnp.float32) + ba_ref[...]
    mu = jnp.mean(t, axis=0, keepdims=True)
    d = t - mu
    var = jnp.mean(d * d, axis=0, keepdims=True)
    hn = d * lax.rsqrt(var + 1e-5) * g_ref[...] + be_ref[...]
    hn = jnp.maximum(hn, 0.0)
    h2 = lax.dot_general(hn, wb_ref[...], (((1,), (1,)), ((), ())),
                         preferred_element_type=jnp.float32) + bb_ref[...]
    o_ref[...] = jnp.maximum(h2, 0.0)


def _head_layer_body(x_ref, p0_ref, p1_ref, wa_ref, ba_ref, g_ref, be_ref,
                     wb_ref, bb_ref, wl1_ref, bl1_ref, wl2_ref, bl2_ref,
                     o_ref):
    h = p0_ref[...] + p1_ref[...] - x_ref[...]
    t = lax.dot_general(h, wa_ref[...], (((1,), (1,)), ((), ())),
                        preferred_element_type=jnp.float32) + ba_ref[...]
    mu = jnp.mean(t, axis=0, keepdims=True)
    d = t - mu
    var = jnp.mean(d * d, axis=0, keepdims=True)
    hn = d * lax.rsqrt(var + 1e-5) * g_ref[...] + be_ref[...]
    hn = jnp.maximum(hn, 0.0)
    h2 = lax.dot_general(hn, wb_ref[...], (((1,), (1,)), ((), ())),
                         preferred_element_type=jnp.float32) + bb_ref[...]
    h2 = jnp.maximum(h2, 0.0)
    h3 = lax.dot_general(h2, wl1_ref[...], (((1,), (1,)), ((), ())),
                         preferred_element_type=jnp.float32) + bl1_ref[...]
    h3 = jnp.maximum(h3, 0.0)
    y = jnp.sum(h3 * wl2_ref[...], axis=1, keepdims=True) + bl2_ref[...]
    o_ref[...] = 1.0 / (1.0 + jnp.exp(-y))


def _gin_dense(x, p0, p1, Wa, ba, g, be, Wb, bb, interpret=False):
    n, f = x.shape
    return pl.pallas_call(
        _dense_layer_body,
        out_shape=jax.ShapeDtypeStruct((n, f), jnp.float32),
        interpret=interpret,
    )(x, p0, p1, Wa, ba.reshape(1, -1), g.reshape(1, -1), be.reshape(1, -1),
      Wb, bb.reshape(1, -1))


def _gin_dense_head(x, p0, p1, Wa, ba, g, be, Wb, bb, Wl1, bl1, Wl2, bl2,
                    interpret=False):
    n, f = x.shape
    return pl.pallas_call(
        _head_layer_body,
        out_shape=jax.ShapeDtypeStruct((n, 1), jnp.float32),
        interpret=interpret,
    )(x, p0, p1, Wa, ba.reshape(1, -1), g.reshape(1, -1), be.reshape(1, -1),
      Wb, bb.reshape(1, -1), Wl1, bl1.reshape(1, -1), Wl2,
      bl2.reshape(1, 1))


def kernel(x, edge_index, W1a, b1a, g1, be1, W1b, b1b, W2a, b2a, g2, be2,
           W2b, b2b, Wl1, bl1, Wl2, bl2):
    n, f = x.shape
    e = edge_index.shape[1]
    chunks = -(-e // (_NW * _CH))
    e_pad = _NW * _CH * chunks
    src = jnp.concatenate(
        [edge_index[0], jnp.zeros((e_pad - e,), jnp.int32)]
    ).reshape(_NW, chunks, _CH)
    dst = jnp.concatenate(
        [edge_index[1], jnp.full((e_pad - e,), n, jnp.int32)]
    ).reshape(_NW, chunks, _CH)

    n_pad = -(-n // (8 * _NS)) * (8 * _NS)
    agg = _make_sc_agg(n_pad, f, chunks)

    def run_agg(nodes):
        nodes_p = jnp.pad(nodes, ((0, n_pad - n), (0, 0)))
        p = agg(nodes_p, src, dst)
        return p[0, :n], p[1, :n]

    p0, p1 = run_agg(x)
    h1 = _gin_dense(x, p0, p1, W1a, b1a, g1, be1, W1b, b1b)
    q0, q1 = run_agg(h1)
    return _gin_dense_head(h1, q0, q1, W2a, b2a, g2, be2, W2b, b2b,
                           Wl1, bl1, Wl2, bl2)


# speedup vs baseline: 1.4564x; 1.2759x over previous
import functools

import jax
import jax.numpy as jnp
from jax import lax
from jax.experimental import pallas as pl
from jax.experimental.pallas import tpu as pltpu
from jax.experimental.pallas import tpu_sc as plsc

_NC = 2
_NS = 16
_NW = _NC * _NS
_CH = 128


@functools.lru_cache(maxsize=None)
def _make_sc_agg(n_pad: int, n_feat: int, cpw0: int, cpw1: int):
    rows_per_tile = n_pad // _NS
    assert rows_per_tile * _NS == n_pad and rows_per_tile % 8 == 0
    chunks = max(cpw0, cpw1)
    mesh = plsc.VectorSubcoreMesh(core_axis_name="c", subcore_axis_name="s")

    @functools.partial(
        pl.kernel,
        out_type=jax.ShapeDtypeStruct((_NC, n_pad, n_feat), jnp.float32),
        mesh=mesh,
        scratch_types=[
            pltpu.VMEM((chunks, _CH), jnp.int32),
            pltpu.VMEM((chunks, _CH), jnp.int32),
            pltpu.VMEM((_CH, n_feat), jnp.float32),
            pltpu.VMEM_SHARED((n_pad, n_feat), jnp.float32),
            pltpu.SemaphoreType.DMA,
        ],
    )
    def agg_kernel(x_hbm, src_hbm, dst_hbm, out_hbm, src_v, dst_v, rows_v,
                   acc, sem):
        c = lax.axis_index("c")
        s = lax.axis_index("s")
        wid = s * _NC + c
        pltpu.sync_copy(src_hbm.at[wid], src_v)
        pltpu.sync_copy(dst_hbm.at[wid], dst_v)
        row0 = s * rows_per_tile
        pltpu.sync_copy(x_hbm.at[pl.ds(row0, rows_per_tile)],
                        acc.at[pl.ds(row0, rows_per_tile)])
        plsc.subcore_barrier()

        def body(j, carry):
            pltpu.async_copy(x_hbm.at[src_v.at[j]], rows_v, sem).wait()
            pltpu.sync_copy(rows_v, acc.at[dst_v.at[j]], add=True)
            return carry

        @pl.when(c == 0)
        def _():
            lax.fori_loop(0, cpw0, body, 0)

        @pl.when(c == 1)
        def _():
            lax.fori_loop(0, cpw1, body, 0)

        plsc.subcore_barrier()
        pltpu.sync_copy(acc.at[pl.ds(row0, rows_per_tile)],
                        out_hbm.at[c, pl.ds(row0, rows_per_tile)])

    return agg_kernel


def _dense_layer_body(x_ref, p0_ref, p1_ref, wa_ref, ba_ref, g_ref, be_ref,
                      wb_ref, bb_ref, o_ref):
    h = p0_ref[...] + p1_ref[...] - x_ref[...]
    t = lax.dot_general(h, wa_ref[...], (((1,), (1,)), ((), ())),
                        preferred_element_type=jnp.float32) + ba_ref[...]
    mu = jnp.mean(t, axis=0, keepdims=True)
    d = t - mu
    var = jnp.mean(d * d, axis=0, keepdims=True)
    hn = d * lax.rsqrt(var + 1e-5) * g_ref[...] + be_ref[...]
    hn = jnp.maximum(hn, 0.0)
    h2 = lax.dot_general(hn, wb_ref[...], (((1,), (1,)), ((), ())),
                         preferred_element_type=jnp.float32) + bb_ref[...]
    o_ref[...] = jnp.maximum(h2, 0.0)


def _head_layer_body(x_ref, p0_ref, p1_ref, wa_ref, ba_ref, g_ref, be_ref,
                     wb_ref, bb_ref, wl1_ref, bl1_ref, wl2_ref, bl2_ref,
                     o_ref):
    h = p0_ref[...] + p1_ref[...] - x_ref[...]
    t = lax.dot_general(h, wa_ref[...], (((1,), (1,)), ((), ())),
                        preferred_element_type=jnp.float32) + ba_ref[...]
    mu = jnp.mean(t, axis=0, keepdims=True)
    d = t - mu
    var = jnp.mean(d * d, axis=0, keepdims=True)
    hn = d * lax.rsqrt(var + 1e-5) * g_ref[...] + be_ref[...]
    hn = jnp.maximum(hn, 0.0)
    h2 = lax.dot_general(hn, wb_ref[...], (((1,), (1,)), ((), ())),
                         preferred_element_type=jnp.float32) + bb_ref[...]
    h2 = jnp.maximum(h2, 0.0)
    h3 = lax.dot_general(h2, wl1_ref[...], (((1,), (1,)), ((), ())),
                         preferred_element_type=jnp.float32) + bl1_ref[...]
    h3 = jnp.maximum(h3, 0.0)
    y = jnp.sum(h3 * wl2_ref[...], axis=1, keepdims=True) + bl2_ref[...]
    o_ref[...] = 1.0 / (1.0 + jnp.exp(-y))


def _gin_dense(x, p0, p1, Wa, ba, g, be, Wb, bb, interpret=False):
    n, f = x.shape
    return pl.pallas_call(
        _dense_layer_body,
        out_shape=jax.ShapeDtypeStruct((n, f), jnp.float32),
        interpret=interpret,
    )(x, p0, p1, Wa, ba.reshape(1, -1), g.reshape(1, -1), be.reshape(1, -1),
      Wb, bb.reshape(1, -1))


def _gin_dense_head(x, p0, p1, Wa, ba, g, be, Wb, bb, Wl1, bl1, Wl2, bl2,
                    interpret=False):
    n, f = x.shape
    return pl.pallas_call(
        _head_layer_body,
        out_shape=jax.ShapeDtypeStruct((n, 1), jnp.float32),
        interpret=interpret,
    )(x, p0, p1, Wa, ba.reshape(1, -1), g.reshape(1, -1), be.reshape(1, -1),
      Wb, bb.reshape(1, -1), Wl1, bl1.reshape(1, -1), Wl2,
      bl2.reshape(1, 1))


def kernel(x, edge_index, W1a, b1a, g1, be1, W1b, b1b, W2a, b2a, g2, be2,
           W2b, b2b, Wl1, bl1, Wl2, bl2):
    n, f = x.shape
    e = edge_index.shape[1]
    cpw0, cpw1 = 52, 105
    assert _NS * _CH * (cpw0 + cpw1) >= e
    e_pad = _NS * _CH * (cpw0 + cpw1)
    chunks = max(cpw0, cpw1)
    e0 = _NS * _CH * cpw0

    def split(idx, fill):
        idx_p = jnp.concatenate(
            [idx, jnp.full((e_pad - e,), fill, jnp.int32)])
        b0 = jnp.pad(idx_p[:e0].reshape(_NS, cpw0, _CH),
                     ((0, 0), (0, chunks - cpw0), (0, 0)),
                     constant_values=fill)
        b1 = jnp.pad(idx_p[e0:].reshape(_NS, cpw1, _CH),
                     ((0, 0), (0, chunks - cpw1), (0, 0)),
                     constant_values=fill)
        return jnp.stack([b0, b1], axis=1).reshape(_NW, chunks, _CH)

    src = split(edge_index[0], 0)
    dst = split(edge_index[1], n)

    n_pad = -(-n // (8 * _NS)) * (8 * _NS)
    agg = _make_sc_agg(n_pad, f, cpw0, cpw1)

    def run_agg(nodes):
        nodes_p = jnp.pad(nodes, ((0, n_pad - n), (0, 0)))
        p = agg(nodes_p, src, dst)
        return p[0, :n], p[1, :n]

    p0, p1 = run_agg(x)
    h1 = _gin_dense(x, p0, p1, W1a, b1a, g1, be1, W1b, b1b)
    q0, q1 = run_agg(h1)
    return _gin_dense_head(h1, q0, q1, W2a, b2a, g2, be2, W2b, b2b,
                           Wl1, bl1, Wl2, bl2)
